# Initial kernel scaffold; baseline (speedup 1.0000x reference)
import jax
import jax.numpy as jnp
from jax.experimental import pallas as pl


def kernel(in_feat, exteraVar1, params, edge_index1, edge_index2, edge_index3, edge_index4, edge_index5):
    raise NotImplementedError("write your pallas kernel here")


# speedup vs baseline: 1.0125x; 1.0125x over previous
import functools

import jax
import jax.numpy as jnp
from jax.experimental import pallas as pl

RES = 128
P = 2
_N = [6 * (RES // (P ** k)) ** 2 for k in range(5)]


def _mm_body(x_ref, w_ref, b_ref, o_ref, *, relu):
    acc = jnp.dot(x_ref[...], w_ref[...], preferred_element_type=jnp.float32)
    if b_ref is not None:
        acc = acc + b_ref[...]
    if relu:
        acc = jnp.maximum(acc, 0.0)
    o_ref[...] = acc


def _mm(x, w, b=None, relu=False, bn=2048):
    n, ci = x.shape
    co = w.shape[1]
    bn = min(bn, n)
    grid = (n // bn,)
    in_specs = [
        pl.BlockSpec((bn, ci), lambda i: (i, 0)),
        pl.BlockSpec((ci, co), lambda i: (0, 0)),
    ]
    args = [x, w]
    if b is not None:
        in_specs.append(pl.BlockSpec((1, co), lambda i: (0, 0)))
        args.append(b.reshape(1, co))
    body = functools.partial(_mm_body, relu=relu)
    if b is None:
        body = lambda x_ref, w_ref, o_ref: _mm_body(x_ref, w_ref, None, o_ref, relu=relu)
    return pl.pallas_call(
        body,
        grid=grid,
        in_specs=in_specs,
        out_specs=pl.BlockSpec((bn, co), lambda i: (i, 0)),
        out_shape=jax.ShapeDtypeStruct((n, co), jnp.float32),
    )(*args)


def _mm2_body(a_ref, b_ref, wa_ref, wb_ref, o_ref):
    acc = jnp.dot(a_ref[...], wa_ref[...], preferred_element_type=jnp.float32)
    acc += jnp.dot(b_ref[...], wb_ref[...], preferred_element_type=jnp.float32)
    o_ref[...] = acc


def _mm2(a, b, wa, wb, bn=2048):
    n, ca = a.shape
    cb = b.shape[1]
    co = wa.shape[1]
    bn = min(bn, n)
    return pl.pallas_call(
        _mm2_body,
        grid=(n // bn,),
        in_specs=[
            pl.BlockSpec((bn, ca), lambda i: (i, 0)),
            pl.BlockSpec((bn, cb), lambda i: (i, 0)),
            pl.BlockSpec((ca, co), lambda i: (0, 0)),
            pl.BlockSpec((cb, co), lambda i: (0, 0)),
        ],
        out_specs=pl.BlockSpec((bn, co), lambda i: (i, 0)),
        out_shape=jax.ShapeDtypeStruct((n, co), jnp.float32),
    )(a, b, wa, wb)


def _combine_body(x_ref, ws_ref, wn_ref, p_ref, dp_ref, b_ref, o_ref, *, relu):
    deg = dp_ref[0, :, 0] + dp_ref[1, :, 0]
    invd = 1.0 / jnp.maximum(deg, 1.0)
    mean = (p_ref[0] + p_ref[1]) * invd[:, None]
    acc = jnp.dot(x_ref[...], ws_ref[...], preferred_element_type=jnp.float32)
    if wn_ref is not None:
        acc += jnp.dot(mean, wn_ref[...], preferred_element_type=jnp.float32)
    else:
        acc += mean
    acc += b_ref[...]
    if relu:
        acc = jnp.maximum(acc, 0.0)
    o_ref[...] = acc


def _combine(x, ws, wn, p, dp, b, relu, bn=2048):
    n, ci = x.shape
    co = ws.shape[1]
    w = p.shape[2]
    bn = min(bn, n)
    in_specs = [
        pl.BlockSpec((bn, ci), lambda i: (i, 0)),
        pl.BlockSpec((ci, co), lambda i: (0, 0)),
    ]
    args = [x, ws]
    if wn is not None:
        in_specs.append(pl.BlockSpec((ci, co), lambda i: (0, 0)))
        args.append(wn)
    in_specs += [
        pl.BlockSpec((2, bn, w), lambda i: (0, i, 0)),
        pl.BlockSpec((2, bn, 16), lambda i: (0, i, 0)),
        pl.BlockSpec((1, co), lambda i: (0, 0)),
    ]
    args += [p, dp, b.reshape(1, co)]

    if wn is not None:
        body = functools.partial(_combine_body, relu=relu)
    else:
        body = lambda x_ref, ws_ref, p_ref, dp_ref, b_ref, o_ref: _combine_body(
            x_ref, ws_ref, None, p_ref, dp_ref, b_ref, o_ref, relu=relu)
    return pl.pallas_call(
        body,
        grid=(n // bn,),
        in_specs=in_specs,
        out_specs=pl.BlockSpec((bn, co), lambda i: (i, 0)),
        out_shape=jax.ShapeDtypeStruct((n, co), jnp.float32),
    )(*args)


def _pool_body(x_ref, o_ref):
    o_ref[...] = jnp.mean(x_ref[...], axis=(1, 3))


def _pool(h, res):
    c = h.shape[1]
    r2 = res // 2
    m = 6 * r2
    x = h.reshape(m, 2, r2, 2, c)
    g = 8 if m % 8 == 0 else 1
    out = pl.pallas_call(
        _pool_body,
        grid=(m // g,),
        in_specs=[pl.BlockSpec((g, 2, r2, 2, c), lambda i: (i, 0, 0, 0, 0))],
        out_specs=pl.BlockSpec((g, r2, c), lambda i: (i, 0, 0)),
        out_shape=jax.ShapeDtypeStruct((m, r2, c), jnp.float32),
    )(x)
    return out.reshape(m * r2, c)


def _up(h, res, w, b):
    c, d = w.shape[0], w.shape[1]
    wr = w.transpose(0, 2, 3, 1).reshape(c, 4 * d)
    b4 = jnp.tile(b, 4)
    p = _mm(h, wr, b4)
    p = p.reshape(6, res, res, 2, 2, d).transpose(0, 1, 3, 2, 4, 5)
    return p.reshape(6 * 4 * res * res, d)


def _seg_partials(y, src, dst, n):
    agg = jax.ops.segment_sum(y[src], dst, num_segments=n)
    return jnp.stack([agg, jnp.zeros_like(agg)])


def _deg_partials(dst, n):
    deg = jax.ops.segment_sum(jnp.ones_like(dst, dtype=jnp.float32), dst, num_segments=n)
    dp = jnp.zeros((2, n, 16), jnp.float32)
    return dp.at[0, :, 0].set(deg)


def _sage(x, src, dst, dp, params, name, n, relu=True):
    ws = params[name + "_Ws"]
    wn = params[name + "_Wn"]
    b = params[name + "_b"]
    ci, co = ws.shape
    if co < ci:
        y = _mm(x, wn)
        p = _seg_partials(y, src, dst, n)
        return _combine(x, ws, None, p, dp, b, relu)
    p = _seg_partials(x, src, dst, n)
    return _combine(x, ws, wn, p, dp, b, relu)


def kernel(in_feat, exteraVar1, params, edge_index1, edge_index2, edge_index3,
           edge_index4, edge_index5):
    del exteraVar1
    edges = [edge_index1, edge_index2, edge_index3, edge_index4, edge_index5]
    srcs = [e[0] for e in edges]
    dsts = [e[1] for e in edges]
    dps = [_deg_partials(dsts[l], _N[l]) for l in range(5)]

    def sage(x, lvl, name, relu=True):
        return _sage(x, srcs[lvl], dsts[lvl], dps[lvl], params, name, _N[lvl], relu)

    h1 = sage(in_feat, 0, "conv1")
    h22 = sage(h1, 0, "conv2")
    h2 = _pool(h22, RES)
    h3 = sage(h2, 1, "conv3")
    h33 = sage(h3, 1, "conv33")
    h3p = _pool(h33, RES // 2)
    h4 = sage(h3p, 2, "conv4")
    h44 = sage(h4, 2, "conv44")
    h4p = _pool(h44, RES // 4)
    h5 = sage(h4p, 3, "conv5")
    h55 = sage(h5, 3, "conv55")
    h5p = _pool(h55, RES // 8)
    h6 = sage(h5p, 4, "conv6")
    h6 = sage(h6, 4, "conv66")
    h6 = sage(h6, 4, "conv7")
    h6 = _up(h6, RES // 16, params["up1_W"], params["up1_b"])
    h6 = jnp.concatenate([h6, h55], axis=1)
    h6 = sage(h6, 3, "conv7")
    h6 = sage(h6, 3, "conv77")
    h6 = sage(h6, 3, "conv8")
    h6 = _up(h6, RES // 8, params["up2_W"], params["up2_b"])
    h6 = jnp.concatenate([h6, h44], axis=1)
    h6 = sage(h6, 2, "conv8")
    h6 = sage(h6, 2, "conv88")
    h6 = sage(h6, 2, "conv9")
    h6 = _up(h6, RES // 4, params["up3_W"], params["up3_b"])
    h6 = jnp.concatenate([h6, h33], axis=1)
    h6 = sage(h6, 1, "conv9")
    h6 = sage(h6, 1, "conv99")
    h6 = sage(h6, 1, "conv10")
    h6 = _up(h6, RES // 2, params["up4_W"], params["up4_b"])
    h6 = jnp.concatenate([h6, h22], axis=1)
    h6 = sage(h6, 0, "conv10")
    h6 = sage(h6, 0, "conv101")
    return sage(h6, 0, "conv11", relu=False)


# speedup vs baseline: 3.7058x; 3.6600x over previous
import functools

import jax
import jax.numpy as jnp
from jax import lax
from jax.experimental import pallas as pl
from jax.experimental.pallas import tpu as pltpu
from jax.experimental.pallas import tpu_sc as plsc

RES = 128
P = 2
_N = [6 * (RES // (P ** k)) ** 2 for k in range(5)]

_SC_CORES = 2
_SC_SUBCORES = 16
_SC_TILES = _SC_CORES * _SC_SUBCORES


def _mm_body(x_ref, w_ref, b_ref, o_ref, *, relu):
    acc = jnp.dot(x_ref[...], w_ref[...], preferred_element_type=jnp.float32)
    if b_ref is not None:
        acc = acc + b_ref[...]
    if relu:
        acc = jnp.maximum(acc, 0.0)
    o_ref[...] = acc


def _mm(x, w, b=None, relu=False, bn=2048):
    n, ci = x.shape
    co = w.shape[1]
    bn = min(bn, n)
    grid = (n // bn,)
    in_specs = [
        pl.BlockSpec((bn, ci), lambda i: (i, 0)),
        pl.BlockSpec((ci, co), lambda i: (0, 0)),
    ]
    args = [x, w]
    if b is not None:
        in_specs.append(pl.BlockSpec((1, co), lambda i: (0, 0)))
        args.append(b.reshape(1, co))
    body = functools.partial(_mm_body, relu=relu)
    if b is None:
        body = lambda x_ref, w_ref, o_ref: _mm_body(x_ref, w_ref, None, o_ref, relu=relu)
    return pl.pallas_call(
        body,
        grid=grid,
        in_specs=in_specs,
        out_specs=pl.BlockSpec((bn, co), lambda i: (i, 0)),
        out_shape=jax.ShapeDtypeStruct((n, co), jnp.float32),
    )(*args)


def _mm2_body(a_ref, b_ref, wa_ref, wb_ref, o_ref):
    acc = jnp.dot(a_ref[...], wa_ref[...], preferred_element_type=jnp.float32)
    acc += jnp.dot(b_ref[...], wb_ref[...], preferred_element_type=jnp.float32)
    o_ref[...] = acc


def _mm2(a, b, wa, wb, bn=2048):
    n, ca = a.shape
    cb = b.shape[1]
    co = wa.shape[1]
    bn = min(bn, n)
    return pl.pallas_call(
        _mm2_body,
        grid=(n // bn,),
        in_specs=[
            pl.BlockSpec((bn, ca), lambda i: (i, 0)),
            pl.BlockSpec((bn, cb), lambda i: (i, 0)),
            pl.BlockSpec((ca, co), lambda i: (0, 0)),
            pl.BlockSpec((cb, co), lambda i: (0, 0)),
        ],
        out_specs=pl.BlockSpec((bn, co), lambda i: (i, 0)),
        out_shape=jax.ShapeDtypeStruct((n, co), jnp.float32),
    )(a, b, wa, wb)


def _combine_body(x_ref, ws_ref, wn_ref, p_ref, dp_ref, b_ref, o_ref, *, relu):
    deg = dp_ref[0, :, 0] + dp_ref[1, :, 0]
    invd = 1.0 / jnp.maximum(deg, 1.0)
    mean = (p_ref[0] + p_ref[1]) * invd[:, None]
    acc = jnp.dot(x_ref[...], ws_ref[...], preferred_element_type=jnp.float32)
    if wn_ref is not None:
        acc += jnp.dot(mean, wn_ref[...], preferred_element_type=jnp.float32)
    else:
        acc += mean
    acc += b_ref[...]
    if relu:
        acc = jnp.maximum(acc, 0.0)
    o_ref[...] = acc


def _combine(x, ws, wn, p, dp, b, relu, bn=2048):
    n, ci = x.shape
    co = ws.shape[1]
    w = p.shape[2]
    bn = min(bn, n)
    in_specs = [
        pl.BlockSpec((bn, ci), lambda i: (i, 0)),
        pl.BlockSpec((ci, co), lambda i: (0, 0)),
    ]
    args = [x, ws]
    if wn is not None:
        in_specs.append(pl.BlockSpec((ci, co), lambda i: (0, 0)))
        args.append(wn)
    in_specs += [
        pl.BlockSpec((2, bn, w), lambda i: (0, i, 0)),
        pl.BlockSpec((2, bn, 16), lambda i: (0, i, 0)),
        pl.BlockSpec((1, co), lambda i: (0, 0)),
    ]
    args += [p, dp, b.reshape(1, co)]

    if wn is not None:
        body = functools.partial(_combine_body, relu=relu)
    else:
        body = lambda x_ref, ws_ref, p_ref, dp_ref, b_ref, o_ref: _combine_body(
            x_ref, ws_ref, None, p_ref, dp_ref, b_ref, o_ref, relu=relu)
    return pl.pallas_call(
        body,
        grid=(n // bn,),
        in_specs=in_specs,
        out_specs=pl.BlockSpec((bn, co), lambda i: (i, 0)),
        out_shape=jax.ShapeDtypeStruct((n, co), jnp.float32),
    )(*args)


def _pool_body(x_ref, o_ref):
    o_ref[...] = jnp.mean(x_ref[...], axis=(1, 3))


def _pool(h, res):
    c = h.shape[1]
    r2 = res // 2
    m = 6 * r2
    x = h.reshape(m, 2, r2, 2, c)
    g = 8 if m % 8 == 0 else 1
    out = pl.pallas_call(
        _pool_body,
        grid=(m // g,),
        in_specs=[pl.BlockSpec((g, 2, r2, 2, c), lambda i: (i, 0, 0, 0, 0))],
        out_specs=pl.BlockSpec((g, r2, c), lambda i: (i, 0, 0)),
        out_shape=jax.ShapeDtypeStruct((m, r2, c), jnp.float32),
    )(x)
    return out.reshape(m * r2, c)


def _up(h, res, w, b):
    c, d = w.shape[0], w.shape[1]
    wr = w.transpose(0, 2, 3, 1).reshape(c, 4 * d)
    b4 = jnp.tile(b, 4)
    p = _mm(h, wr, b4)
    p = p.reshape(6, res, res, 2, 2, d).transpose(0, 1, 3, 2, 4, 5)
    return p.reshape(6 * 4 * res * res, d)


def _seg_group_size(e_tile):
    for g in range(min(128, e_tile), 0, -8):
        if e_tile % g == 0:
            return g
    raise ValueError(e_tile)


@functools.lru_cache(maxsize=None)
def _make_seg_kernel(n, e, w, n_passes, ones_mode):
    half = n // n_passes
    half_pad = half + 128
    zstripe = half_pad // _SC_SUBCORES
    stripe = half // _SC_SUBCORES
    e_tile = e // _SC_TILES
    g = _seg_group_size(e_tile)
    n_groups = e_tile // g
    mesh = plsc.VectorSubcoreMesh(core_axis_name="c", subcore_axis_name="s")

    def body(*refs):
        if ones_mode:
            ones_hbm, *dstls, zeros_hbm, out_hbm, accum, dstv, rows, sem = refs
            src_hbm = srcv = None
        else:
            y_hbm, src_hbm, *dstls, zeros_hbm, out_hbm, accum, srcv, dstv, rows, sem = refs
        c = lax.axis_index("c")
        s = lax.axis_index("s")
        tid = c * _SC_SUBCORES + s
        ebase = tid * e_tile
        if ones_mode:
            pltpu.sync_copy(ones_hbm, rows)
        for p in range(n_passes):
            pltpu.sync_copy(zeros_hbm, accum.at[pl.ds(s * zstripe, zstripe)])
            plsc.subcore_barrier()
            dstl = dstls[p]

            def grp(j, carry):
                off = ebase + j * g
                pltpu.sync_copy(dstl.at[pl.ds(off, g)], dstv)
                if not ones_mode:
                    pltpu.sync_copy(src_hbm.at[pl.ds(off, g)], srcv)
                    pltpu.async_copy(y_hbm.at[srcv], rows, sem).wait()
                pltpu.sync_copy(rows, accum.at[dstv], add=True)
                return carry

            lax.fori_loop(0, n_groups, grp, 0)
            plsc.subcore_barrier()
            pltpu.sync_copy(
                accum.at[pl.ds(s * stripe, stripe)],
                out_hbm.at[pl.ds(c * n + p * half + s * stripe, stripe)])
            if p + 1 < n_passes:
                plsc.subcore_barrier()

    scratch = []
    if not ones_mode:
        scratch.append(pltpu.VMEM((g,), jnp.int32))
    scratch += [
        pltpu.VMEM((g,), jnp.int32),
        pltpu.VMEM((g, w), jnp.float32),
        pltpu.SemaphoreType.DMA,
    ]
    return pl.kernel(
        body,
        out_type=jax.ShapeDtypeStruct((_SC_CORES * n, w), jnp.float32),
        mesh=mesh,
        scratch_types=[pltpu.VMEM_SHARED((half_pad, w), jnp.float32)] + scratch,
        compiler_params=pltpu.CompilerParams(use_tc_tiling_on_sc=False),
    )


def _seg_partials(y, src, dstls, n):
    w = y.shape[1]
    e = src.shape[0]
    half_pad = (n // len(dstls)) + 128
    zeros = jnp.zeros((half_pad // _SC_SUBCORES, w), jnp.float32)
    kfn = _make_seg_kernel(n, e, w, len(dstls), False)
    out = kfn(y, src, *dstls, zeros)
    return out.reshape(_SC_CORES, n, w)


def _deg_partials(dstls, e, n):
    e_tile = e // _SC_TILES
    g = _seg_group_size(e_tile)
    ones = jnp.ones((g, 16), jnp.float32)
    half_pad = (n // len(dstls)) + 128
    zeros = jnp.zeros((half_pad // _SC_SUBCORES, 16), jnp.float32)
    kfn = _make_seg_kernel(n, e, 16, len(dstls), True)
    out = kfn(ones, *dstls, zeros)
    return out.reshape(_SC_CORES, n, 16)


def _dst_split_body(d_ref, lo_ref, hi_ref, *, half):
    d = d_ref[...]
    lo_ref[...] = jnp.where(d < half, d, half)
    hi_ref[...] = jnp.where(d >= half, d - half, half)


def _dst_split(dst, half):
    e = dst.shape[0]
    rows = e // 128
    x = dst.reshape(rows, 128)
    br = 512
    body = functools.partial(_dst_split_body, half=half)
    lo, hi = pl.pallas_call(
        body,
        grid=(rows // br,),
        in_specs=[pl.BlockSpec((br, 128), lambda i: (i, 0))],
        out_specs=[pl.BlockSpec((br, 128), lambda i: (i, 0))] * 2,
        out_shape=[jax.ShapeDtypeStruct((rows, 128), jnp.int32)] * 2,
    )(x)
    return lo.reshape(e), hi.reshape(e)


_SPMEM_BUDGET = 7 * 1024 * 1024


def _num_passes(n, w):
    p = 1
    while (n // p + 128) * w * 4 > _SPMEM_BUDGET:
        p *= 2
    return p


def _sage(x, src, dstls_fn, dp, params, name, n, relu=True):
    ws = params[name + "_Ws"]
    wn = params[name + "_Wn"]
    b = params[name + "_b"]
    ci, co = ws.shape
    w = min(ci, co)
    dstls = dstls_fn(_num_passes(n, w))
    if co < ci:
        y = _mm(x, wn)
        p = _seg_partials(y, src, dstls, n)
        return _combine(x, ws, None, p, dp, b, relu)
    p = _seg_partials(x, src, dstls, n)
    return _combine(x, ws, wn, p, dp, b, relu)


def kernel(in_feat, exteraVar1, params, edge_index1, edge_index2, edge_index3,
           edge_index4, edge_index5):
    del exteraVar1
    edges = [edge_index1, edge_index2, edge_index3, edge_index4, edge_index5]
    srcs = [e[0] for e in edges]
    dsts = [e[1] for e in edges]
    split_cache = {}

    def dstls_fn(lvl):
        def get(n_passes):
            if n_passes == 1:
                return [dsts[lvl]]
            key = (lvl, n_passes)
            if key not in split_cache:
                assert n_passes == 2
                split_cache[key] = list(_dst_split(dsts[lvl], _N[lvl] // 2))
            return split_cache[key]
        return get

    fns = [dstls_fn(l) for l in range(5)]
    dps = [_deg_partials(fns[l](_num_passes(_N[l], 16)), dsts[l].shape[0], _N[l])
           for l in range(5)]

    def sage(x, lvl, name, relu=True):
        return _sage(x, srcs[lvl], fns[lvl], dps[lvl], params, name, _N[lvl], relu)

    h1 = sage(in_feat, 0, "conv1")
    h22 = sage(h1, 0, "conv2")
    h2 = _pool(h22, RES)
    h3 = sage(h2, 1, "conv3")
    h33 = sage(h3, 1, "conv33")
    h3p = _pool(h33, RES // 2)
    h4 = sage(h3p, 2, "conv4")
    h44 = sage(h4, 2, "conv44")
    h4p = _pool(h44, RES // 4)
    h5 = sage(h4p, 3, "conv5")
    h55 = sage(h5, 3, "conv55")
    h5p = _pool(h55, RES // 8)
    h6 = sage(h5p, 4, "conv6")
    h6 = sage(h6, 4, "conv66")
    h6 = sage(h6, 4, "conv7")
    h6 = _up(h6, RES // 16, params["up1_W"], params["up1_b"])
    h6 = jnp.concatenate([h6, h55], axis=1)
    h6 = sage(h6, 3, "conv7")
    h6 = sage(h6, 3, "conv77")
    h6 = sage(h6, 3, "conv8")
    h6 = _up(h6, RES // 8, params["up2_W"], params["up2_b"])
    h6 = jnp.concatenate([h6, h44], axis=1)
    h6 = sage(h6, 2, "conv8")
    h6 = sage(h6, 2, "conv88")
    h6 = sage(h6, 2, "conv9")
    h6 = _up(h6, RES // 4, params["up3_W"], params["up3_b"])
    h6 = jnp.concatenate([h6, h33], axis=1)
    h6 = sage(h6, 1, "conv9")
    h6 = sage(h6, 1, "conv99")
    h6 = sage(h6, 1, "conv10")
    h6 = _up(h6, RES // 2, params["up4_W"], params["up4_b"])
    h6 = jnp.concatenate([h6, h22], axis=1)
    h6 = sage(h6, 0, "conv10")
    h6 = sage(h6, 0, "conv101")
    return sage(h6, 0, "conv11", relu=False)


# speedup vs baseline: 4.6326x; 1.2501x over previous
import functools

import jax
import jax.numpy as jnp
from jax import lax
from jax.experimental import pallas as pl
from jax.experimental.pallas import tpu as pltpu
from jax.experimental.pallas import tpu_sc as plsc

RES = 128
P = 2
_N = [6 * (RES // (P ** k)) ** 2 for k in range(5)]

_SC_CORES = 2
_SC_SUBCORES = 16
_SC_TILES = _SC_CORES * _SC_SUBCORES


def _mm_body(x_ref, w_ref, b_ref, o_ref, *, relu):
    acc = jnp.dot(x_ref[...], w_ref[...], preferred_element_type=jnp.float32)
    if b_ref is not None:
        acc = acc + b_ref[...]
    if relu:
        acc = jnp.maximum(acc, 0.0)
    o_ref[...] = acc


def _mm(x, w, b=None, relu=False, bn=2048):
    n, ci = x.shape
    co = w.shape[1]
    bn = min(bn, n)
    grid = (n // bn,)
    in_specs = [
        pl.BlockSpec((bn, ci), lambda i: (i, 0)),
        pl.BlockSpec((ci, co), lambda i: (0, 0)),
    ]
    args = [x, w]
    if b is not None:
        in_specs.append(pl.BlockSpec((1, co), lambda i: (0, 0)))
        args.append(b.reshape(1, co))
    body = functools.partial(_mm_body, relu=relu)
    if b is None:
        body = lambda x_ref, w_ref, o_ref: _mm_body(x_ref, w_ref, None, o_ref, relu=relu)
    return pl.pallas_call(
        body,
        grid=grid,
        in_specs=in_specs,
        out_specs=pl.BlockSpec((bn, co), lambda i: (i, 0)),
        out_shape=jax.ShapeDtypeStruct((n, co), jnp.float32),
    )(*args)


def _mm2_body(a_ref, b_ref, wa_ref, wb_ref, o_ref):
    acc = jnp.dot(a_ref[...], wa_ref[...], preferred_element_type=jnp.float32)
    acc += jnp.dot(b_ref[...], wb_ref[...], preferred_element_type=jnp.float32)
    o_ref[...] = acc


def _mm2(a, b, wa, wb, bn=2048):
    n, ca = a.shape
    cb = b.shape[1]
    co = wa.shape[1]
    bn = min(bn, n)
    return pl.pallas_call(
        _mm2_body,
        grid=(n // bn,),
        in_specs=[
            pl.BlockSpec((bn, ca), lambda i: (i, 0)),
            pl.BlockSpec((bn, cb), lambda i: (i, 0)),
            pl.BlockSpec((ca, co), lambda i: (0, 0)),
            pl.BlockSpec((cb, co), lambda i: (0, 0)),
        ],
        out_specs=pl.BlockSpec((bn, co), lambda i: (i, 0)),
        out_shape=jax.ShapeDtypeStruct((n, co), jnp.float32),
    )(a, b, wa, wb)


def _combine_body(x_ref, ws_ref, wn_ref, p_ref, dp_ref, b_ref, o_ref, *, relu):
    deg = dp_ref[0, :, 0] + dp_ref[1, :, 0]
    invd = 1.0 / jnp.maximum(deg, 1.0)
    mean = (p_ref[0] + p_ref[1]) * invd[:, None]
    acc = jnp.dot(x_ref[...], ws_ref[...], preferred_element_type=jnp.float32)
    if wn_ref is not None:
        acc += jnp.dot(mean, wn_ref[...], preferred_element_type=jnp.float32)
    else:
        acc += mean
    acc += b_ref[...]
    if relu:
        acc = jnp.maximum(acc, 0.0)
    o_ref[...] = acc


def _combine(x, ws, wn, p, dp, b, relu, bn=2048):
    n, ci = x.shape
    co = ws.shape[1]
    w = p.shape[2]
    bn = min(bn, n)
    in_specs = [
        pl.BlockSpec((bn, ci), lambda i: (i, 0)),
        pl.BlockSpec((ci, co), lambda i: (0, 0)),
    ]
    args = [x, ws]
    if wn is not None:
        in_specs.append(pl.BlockSpec((ci, co), lambda i: (0, 0)))
        args.append(wn)
    in_specs += [
        pl.BlockSpec((2, bn, w), lambda i: (0, i, 0)),
        pl.BlockSpec((2, bn, 16), lambda i: (0, i, 0)),
        pl.BlockSpec((1, co), lambda i: (0, 0)),
    ]
    args += [p, dp, b.reshape(1, co)]

    if wn is not None:
        body = functools.partial(_combine_body, relu=relu)
    else:
        body = lambda x_ref, ws_ref, p_ref, dp_ref, b_ref, o_ref: _combine_body(
            x_ref, ws_ref, None, p_ref, dp_ref, b_ref, o_ref, relu=relu)
    return pl.pallas_call(
        body,
        grid=(n // bn,),
        in_specs=in_specs,
        out_specs=pl.BlockSpec((bn, co), lambda i: (i, 0)),
        out_shape=jax.ShapeDtypeStruct((n, co), jnp.float32),
    )(*args)


def _pool_body(x_ref, o_ref):
    o_ref[...] = jnp.mean(x_ref[...], axis=(1, 3))


def _pool(h, res):
    c = h.shape[1]
    r2 = res // 2
    m = 6 * r2
    x = h.reshape(m, 2, r2, 2, c)
    g = 8 if m % 8 == 0 else 1
    out = pl.pallas_call(
        _pool_body,
        grid=(m // g,),
        in_specs=[pl.BlockSpec((g, 2, r2, 2, c), lambda i: (i, 0, 0, 0, 0))],
        out_specs=pl.BlockSpec((g, r2, c), lambda i: (i, 0, 0)),
        out_shape=jax.ShapeDtypeStruct((m, r2, c), jnp.float32),
    )(x)
    return out.reshape(m * r2, c)


def _up(h, res, w, b):
    c, d = w.shape[0], w.shape[1]
    wr = w.transpose(0, 2, 3, 1).reshape(c, 4 * d)
    b4 = jnp.tile(b, 4)
    p = _mm(h, wr, b4)
    p = p.reshape(6, res, res, 2, 2, d).transpose(0, 1, 3, 2, 4, 5)
    return p.reshape(6 * 4 * res * res, d)


def _seg_group_size(e_tile):
    for g in range(min(128, e_tile), 0, -8):
        if e_tile % g == 0:
            return g
    raise ValueError(e_tile)


def _seg_config(e, n, w, n_passes):
    e_tile = e // _SC_TILES
    accum_bytes = (n // n_passes + 128) * w * 4
    g = _seg_group_size(e_tile)
    while True:
        ng = e_tile // g
        nc = max(d for d in range(1, min(ng, 32) + 1) if ng % d == 0)
        tile_bytes = 2 * nc * g * 4 + 2 * g * w * 4
        if accum_bytes + _SC_SUBCORES * tile_bytes <= int(7.6 * 1024 * 1024):
            return g, nc
        ng2 = g
        for g2 in range(g - 8, 0, -8):
            if e_tile % g2 == 0:
                ng2 = g2
                break
        if ng2 == g:
            return g, nc
        g = ng2


@functools.lru_cache(maxsize=None)
def _make_seg_kernel(n, e, w, n_passes, ones_mode):
    half = n // n_passes
    half_pad = half + 128
    zstripe = half // _SC_SUBCORES
    stripe = half // _SC_SUBCORES
    e_tile = e // _SC_TILES
    g, nc = _seg_config(e, n, w, n_passes)
    ng = e_tile // g
    n_chunks = ng // nc
    pipe = nc >= 2 and not ones_mode
    mesh = plsc.VectorSubcoreMesh(core_axis_name="c", subcore_axis_name="s")

    def body(*refs):
        if ones_mode:
            ones_hbm, *dstls, zeros_hbm, out_hbm, accum, dstv, rows = refs
            src_hbm = srcv = sems = None
        else:
            (y_hbm, src_hbm, *dstls, zeros_hbm, out_hbm,
             accum, srcv, dstv, rows, sem0, sem1) = refs
            sems = (sem0, sem1)
        c = lax.axis_index("c")
        s = lax.axis_index("s")
        tid = c * _SC_SUBCORES + s
        if ones_mode:
            pltpu.sync_copy(ones_hbm, rows)
        for p in range(n_passes):
            pltpu.sync_copy(zeros_hbm, accum.at[pl.ds(s * zstripe, zstripe)])
            plsc.subcore_barrier()
            dstl = dstls[p]

            def chunkbody(ch, carry):
                gbase = tid * ng + ch * nc
                pltpu.sync_copy(dstl.at[pl.ds(gbase, nc)], dstv)
                if ones_mode:
                    def grp(j, cc):
                        pltpu.sync_copy(rows, accum.at[dstv.at[j]], add=True)
                        return cc
                    lax.fori_loop(0, nc, grp, 0)
                elif not pipe:
                    pltpu.sync_copy(src_hbm.at[pl.ds(gbase, nc)], srcv)

                    def grp(j, cc):
                        pltpu.async_copy(y_hbm.at[srcv.at[j]], rows.at[0],
                                         sems[0]).wait()
                        pltpu.sync_copy(rows.at[0], accum.at[dstv.at[j]],
                                        add=True)
                        return cc
                    lax.fori_loop(0, nc, grp, 0)
                else:
                    pltpu.sync_copy(src_hbm.at[pl.ds(gbase, nc)], srcv)
                    for b in range(2):
                        pltpu.async_copy(y_hbm.at[srcv.at[b]], rows.at[b],
                                         sems[b])

                    def pairbody(i, cc):
                        for b in range(2):
                            j = 2 * i + b

                            @pl.when(j < nc)
                            def _():
                                pltpu.make_async_copy(
                                    y_hbm.at[srcv.at[0]], rows.at[b],
                                    sems[b]).wait()
                                pltpu.sync_copy(rows.at[b],
                                                accum.at[dstv.at[j]], add=True)

                                @pl.when(j + 2 < nc)
                                def _():
                                    pltpu.async_copy(y_hbm.at[srcv.at[j + 2]],
                                                     rows.at[b], sems[b])
                        return cc

                    lax.fori_loop(0, (nc + 1) // 2, pairbody, 0)
                return carry

            lax.fori_loop(0, n_chunks, chunkbody, 0)
            plsc.subcore_barrier()
            pltpu.sync_copy(
                accum.at[pl.ds(s * stripe, stripe)],
                out_hbm.at[pl.ds(c * n + p * half + s * stripe, stripe)])
            if p + 1 < n_passes:
                plsc.subcore_barrier()

    scratch = [pltpu.VMEM_SHARED((half_pad, w), jnp.float32)]
    if not ones_mode:
        scratch.append(pltpu.VMEM((nc, g), jnp.int32))
    scratch.append(pltpu.VMEM((nc, g), jnp.int32))
    if ones_mode:
        scratch.append(pltpu.VMEM((g, w), jnp.float32))
    else:
        scratch += [
            pltpu.VMEM((2, g, w), jnp.float32),
            pltpu.SemaphoreType.DMA,
            pltpu.SemaphoreType.DMA,
        ]
    return pl.kernel(
        body,
        out_type=jax.ShapeDtypeStruct((_SC_CORES * n, w), jnp.float32),
        mesh=mesh,
        scratch_types=scratch,
        compiler_params=pltpu.CompilerParams(use_tc_tiling_on_sc=False),
    )


def _seg_partials(y, src, dstls, n):
    w = y.shape[1]
    e = src.shape[0]
    g, _ = _seg_config(e, n, w, len(dstls))
    zeros = jnp.zeros((n // len(dstls) // _SC_SUBCORES, w), jnp.float32)
    kfn = _make_seg_kernel(n, e, w, len(dstls), False)
    out = kfn(y, src.reshape(e // g, g), *[d.reshape(e // g, g) for d in dstls],
              zeros)
    return out.reshape(_SC_CORES, n, w)


def _deg_partials(dstls, e, n):
    g, _ = _seg_config(e, n, 16, len(dstls))
    ones = jnp.ones((g, 16), jnp.float32)
    zeros = jnp.zeros((n // len(dstls) // _SC_SUBCORES, 16), jnp.float32)
    kfn = _make_seg_kernel(n, e, 16, len(dstls), True)
    out = kfn(ones, *[d.reshape(e // g, g) for d in dstls], zeros)
    return out.reshape(_SC_CORES, n, 16)


def _dst_split_body(d_ref, lo_ref, hi_ref, *, half):
    d = d_ref[...]
    lo_ref[...] = jnp.where(d < half, d, half)
    hi_ref[...] = jnp.where(d >= half, d - half, half)


def _dst_split(dst, half):
    e = dst.shape[0]
    rows = e // 128
    x = dst.reshape(rows, 128)
    br = 512
    body = functools.partial(_dst_split_body, half=half)
    lo, hi = pl.pallas_call(
        body,
        grid=(rows // br,),
        in_specs=[pl.BlockSpec((br, 128), lambda i: (i, 0))],
        out_specs=[pl.BlockSpec((br, 128), lambda i: (i, 0))] * 2,
        out_shape=[jax.ShapeDtypeStruct((rows, 128), jnp.int32)] * 2,
    )(x)
    return lo.reshape(e), hi.reshape(e)


_SPMEM_BUDGET = 7 * 1024 * 1024


def _num_passes(n, w):
    p = 1
    while (n // p + 128) * w * 4 > _SPMEM_BUDGET:
        p *= 2
    return p


def _sage(x, src, dstls_fn, dp, params, name, n, relu=True):
    ws = params[name + "_Ws"]
    wn = params[name + "_Wn"]
    b = params[name + "_b"]
    ci, co = ws.shape
    w = min(ci, co)
    dstls = dstls_fn(_num_passes(n, w))
    if co < ci:
        y = _mm(x, wn)
        p = _seg_partials(y, src, dstls, n)
        return _combine(x, ws, None, p, dp, b, relu)
    p = _seg_partials(x, src, dstls, n)
    return _combine(x, ws, wn, p, dp, b, relu)


def kernel(in_feat, exteraVar1, params, edge_index1, edge_index2, edge_index3,
           edge_index4, edge_index5):
    del exteraVar1
    edges = [edge_index1, edge_index2, edge_index3, edge_index4, edge_index5]
    srcs = [e[0] for e in edges]
    dsts = [e[1] for e in edges]
    split_cache = {}

    def dstls_fn(lvl):
        def get(n_passes):
            if n_passes == 1:
                return [dsts[lvl]]
            key = (lvl, n_passes)
            if key not in split_cache:
                assert n_passes == 2
                split_cache[key] = list(_dst_split(dsts[lvl], _N[lvl] // 2))
            return split_cache[key]
        return get

    fns = [dstls_fn(l) for l in range(5)]
    dps = [_deg_partials(fns[l](_num_passes(_N[l], 16)), dsts[l].shape[0], _N[l])
           for l in range(5)]

    def sage(x, lvl, name, relu=True):
        return _sage(x, srcs[lvl], fns[lvl], dps[lvl], params, name, _N[lvl], relu)

    h1 = sage(in_feat, 0, "conv1")
    h22 = sage(h1, 0, "conv2")
    h2 = _pool(h22, RES)
    h3 = sage(h2, 1, "conv3")
    h33 = sage(h3, 1, "conv33")
    h3p = _pool(h33, RES // 2)
    h4 = sage(h3p, 2, "conv4")
    h44 = sage(h4, 2, "conv44")
    h4p = _pool(h44, RES // 4)
    h5 = sage(h4p, 3, "conv5")
    h55 = sage(h5, 3, "conv55")
    h5p = _pool(h55, RES // 8)
    h6 = sage(h5p, 4, "conv6")
    h6 = sage(h6, 4, "conv66")
    h6 = sage(h6, 4, "conv7")
    h6 = _up(h6, RES // 16, params["up1_W"], params["up1_b"])
    h6 = jnp.concatenate([h6, h55], axis=1)
    h6 = sage(h6, 3, "conv7")
    h6 = sage(h6, 3, "conv77")
    h6 = sage(h6, 3, "conv8")
    h6 = _up(h6, RES // 8, params["up2_W"], params["up2_b"])
    h6 = jnp.concatenate([h6, h44], axis=1)
    h6 = sage(h6, 2, "conv8")
    h6 = sage(h6, 2, "conv88")
    h6 = sage(h6, 2, "conv9")
    h6 = _up(h6, RES // 4, params["up3_W"], params["up3_b"])
    h6 = jnp.concatenate([h6, h33], axis=1)
    h6 = sage(h6, 1, "conv9")
    h6 = sage(h6, 1, "conv99")
    h6 = sage(h6, 1, "conv10")
    h6 = _up(h6, RES // 2, params["up4_W"], params["up4_b"])
    h6 = jnp.concatenate([h6, h22], axis=1)
    h6 = sage(h6, 0, "conv10")
    h6 = sage(h6, 0, "conv101")
    return sage(h6, 0, "conv11", relu=False)
